# baseline (speedup 1.0000x reference)
import jax
import jax.numpy as jnp
from jax import lax
from jax.experimental import pallas as pl

INPUT_DIM = 512
TIME_STEPS = 100
MAX_LAT = 99
BT = 8
NLANE = 128


def _body(x_ref, o_ref):
    xv = x_ref[...]
    s = jax.nn.sigmoid(xv)
    latf = jnp.clip(jnp.round((1.0 - s) * float(MAX_LAT)), 0.0, float(MAX_LAT))
    bsel = lax.broadcasted_iota(jnp.int32, (BT, BT * NLANE), 1) // NLANE
    wsel = jnp.where(bsel == lax.broadcasted_iota(jnp.int32, (BT, BT * NLANE), 0),
                     1.0, 0.0).astype(jnp.bfloat16)
    latb = lax.dot_general(latf.astype(jnp.bfloat16), wsel,
                           (((0,), (0,)), ((), ())),
                           preferred_element_type=jnp.float32)
    t = lax.broadcasted_iota(jnp.int32, (INPUT_DIM, BT * NLANE), 1) % NLANE
    hit = (latb == t.astype(jnp.float32)) & (t <= (MAX_LAT - 1) // 2)
    oneh = jnp.where(hit, 1.0, 0.0).astype(jnp.bfloat16)
    for b in range(BT):
        o_ref[b] = oneh[:, b * NLANE:(b + 1) * NLANE]


def kernel(x):
    B, F = x.shape
    y = pl.pallas_call(
        _body,
        grid=(B // BT,),
        in_specs=[pl.BlockSpec((BT, F), lambda i: (i, 0))],
        out_specs=pl.BlockSpec((BT, F, NLANE), lambda i: (i, 0, 0)),
        out_shape=jax.ShapeDtypeStruct((B, F, NLANE), jnp.bfloat16),
    )(x)
    return y[:, :, :TIME_STEPS].astype(jnp.float32)


# speedup vs baseline: 1.5243x; 1.5243x over previous
import jax
import jax.numpy as jnp
from jax import lax
from jax.experimental import pallas as pl

INPUT_DIM = 512
TIME_STEPS = 100
MAX_LAT = 99
BT = 16
NLANE = 128


def _body(x_ref, o_ref):
    xv = x_ref[...]
    s = jax.nn.sigmoid(xv)
    latf = jnp.clip(jnp.round((1.0 - s) * float(MAX_LAT)), 0.0, float(MAX_LAT))
    bsel = lax.broadcasted_iota(jnp.int32, (BT, BT * NLANE), 1) // NLANE
    wsel = jnp.where(bsel == lax.broadcasted_iota(jnp.int32, (BT, BT * NLANE), 0),
                     1.0, 0.0).astype(jnp.bfloat16)
    latb = lax.dot_general(latf.astype(jnp.bfloat16), wsel,
                           (((0,), (0,)), ((), ())),
                           preferred_element_type=jnp.float32)
    t = lax.broadcasted_iota(jnp.int32, (INPUT_DIM, BT * NLANE), 1) % NLANE
    hit = (latb == t.astype(jnp.float32)) & (t <= (MAX_LAT - 1) // 2)
    oneh = jnp.where(hit, 1.0, 0.0).astype(jnp.float32)
    for b in range(BT):
        o_ref[b] = oneh[:, b * NLANE:(b + 1) * NLANE]


def kernel(x):
    B, F = x.shape
    y = pl.pallas_call(
        _body,
        grid=(B // BT,),
        in_specs=[pl.BlockSpec((BT, F), lambda i: (i, 0))],
        out_specs=pl.BlockSpec((BT, F, NLANE), lambda i: (i, 0, 0)),
        out_shape=jax.ShapeDtypeStruct((B, F, NLANE), jnp.float32),
    )(x)
    return y[:, :, :TIME_STEPS]


# speedup vs baseline: 1.5658x; 1.0272x over previous
import jax
import jax.numpy as jnp
from jax import lax
from jax.experimental import pallas as pl

INPUT_DIM = 512
TIME_STEPS = 100
MAX_LAT = 99
BT = 32
NLANE = 128


def _body(x_ref, o_ref):
    xv = x_ref[...]
    s = jax.nn.sigmoid(xv)
    latf = jnp.clip(jnp.round((1.0 - s) * float(MAX_LAT)), 0.0, float(MAX_LAT))
    bsel = lax.broadcasted_iota(jnp.int32, (BT, BT * NLANE), 1) // NLANE
    wsel = jnp.where(bsel == lax.broadcasted_iota(jnp.int32, (BT, BT * NLANE), 0),
                     1.0, 0.0).astype(jnp.bfloat16)
    latb = lax.dot_general(latf.astype(jnp.bfloat16), wsel,
                           (((0,), (0,)), ((), ())),
                           preferred_element_type=jnp.float32)
    t = lax.broadcasted_iota(jnp.int32, (INPUT_DIM, BT * NLANE), 1) % NLANE
    hit = (latb == t.astype(jnp.float32)) & (t <= (MAX_LAT - 1) // 2)
    oneh = jnp.where(hit, 1.0, 0.0).astype(jnp.float32)
    for b in range(BT):
        o_ref[b] = oneh[:, b * NLANE:(b + 1) * NLANE]


def kernel(x):
    B, F = x.shape
    y = pl.pallas_call(
        _body,
        grid=(B // BT,),
        in_specs=[pl.BlockSpec((BT, F), lambda i: (i, 0))],
        out_specs=pl.BlockSpec((BT, F, NLANE), lambda i: (i, 0, 0)),
        out_shape=jax.ShapeDtypeStruct((B, F, NLANE), jnp.float32),
    )(x)
    return y[:, :, :TIME_STEPS]
